# baseline (speedup 1.0000x reference)
import jax
import jax.numpy as jnp
from jax.experimental import pallas as pl


def _fused_kernel(eut_ref, evt_ref, ut_ref, vt_ref, pt_ref, qt_ref):
    pt_ref[...] = jnp.dot(eut_ref[...], ut_ref[...],
                          preferred_element_type=jnp.float32)
    qt_ref[...] = jnp.dot(evt_ref[...], vt_ref[...],
                          preferred_element_type=jnp.float32)


def kernel(U, V, Eu, Ev):
    Ut, Vt, EuT, EvT = U.T, V.T, Eu.T, Ev.T
    d, nu = Ut.shape
    _, nv = Vt.shape
    k = EuT.shape[0]
    steps = 32
    bu = nu // steps
    bv = nv // steps
    Pt, Qt = pl.pallas_call(
        _fused_kernel,
        grid=(steps,),
        in_specs=[
            pl.BlockSpec((k, d), lambda i: (0, 0)),
            pl.BlockSpec((k, d), lambda i: (0, 0)),
            pl.BlockSpec((d, bu), lambda i: (0, i)),
            pl.BlockSpec((d, bv), lambda i: (0, i)),
        ],
        out_specs=[
            pl.BlockSpec((k, bu), lambda i: (0, i)),
            pl.BlockSpec((k, bv), lambda i: (0, i)),
        ],
        out_shape=[
            jax.ShapeDtypeStruct((k, nu), jnp.float32),
            jax.ShapeDtypeStruct((k, nv), jnp.float32),
        ],
    )(EuT, EvT, Ut, Vt)
    return (Pt.T, Qt.T)


# speedup vs baseline: 1.1897x; 1.1897x over previous
import jax
import jax.numpy as jnp
from jax.experimental import pallas as pl


def _fused_kernel(eut_ref, evt_ref, ut_ref, vt_ref, pt_ref, qt_ref):
    pt_ref[...] = jnp.dot(eut_ref[...], ut_ref[...],
                          preferred_element_type=jnp.float32)
    qt_ref[...] = jnp.dot(evt_ref[...], vt_ref[...],
                          preferred_element_type=jnp.float32)


def kernel(U, V, Eu, Ev):
    Ut, Vt, EuT, EvT = U.T, V.T, Eu.T, Ev.T
    d, nu = Ut.shape
    _, nv = Vt.shape
    k = EuT.shape[0]
    steps = 16
    bu = nu // steps
    bv = nv // (steps // 2)
    Pt, Qt = pl.pallas_call(
        _fused_kernel,
        grid=(steps,),
        in_specs=[
            pl.BlockSpec((k, d), lambda i: (0, 0)),
            pl.BlockSpec((k, d), lambda i: (0, 0)),
            pl.BlockSpec((d, bu), lambda i: (0, i)),
            pl.BlockSpec((d, bv), lambda i: (0, i // 2)),
        ],
        out_specs=[
            pl.BlockSpec((k, bu), lambda i: (0, i)),
            pl.BlockSpec((k, bv), lambda i: (0, i // 2)),
        ],
        out_shape=[
            jax.ShapeDtypeStruct((k, nu), jnp.float32),
            jax.ShapeDtypeStruct((k, nv), jnp.float32),
        ],
    )(EuT, EvT, Ut, Vt)
    return (Pt.T, Qt.T)


# speedup vs baseline: 1.2701x; 1.0676x over previous
import jax
import jax.numpy as jnp
from jax.experimental import pallas as pl


def _fused_kernel(eut_ref, evt_ref, ua_ref, ub_ref, va_ref, vb_ref,
                  pt_ref, qt_ref):
    bu = ua_ref.shape[1]
    bv = va_ref.shape[1]
    pt_ref[:, :bu] = jnp.dot(eut_ref[...], ua_ref[...],
                             preferred_element_type=jnp.float32)
    pt_ref[:, bu:] = jnp.dot(eut_ref[...], ub_ref[...],
                             preferred_element_type=jnp.float32)
    qt_ref[:, :bv] = jnp.dot(evt_ref[...], va_ref[...],
                             preferred_element_type=jnp.float32)
    qt_ref[:, bv:] = jnp.dot(evt_ref[...], vb_ref[...],
                             preferred_element_type=jnp.float32)


def kernel(U, V, Eu, Ev):
    Ut, Vt, EuT, EvT = U.T, V.T, Eu.T, Ev.T
    d, nu = Ut.shape
    _, nv = Vt.shape
    k = EuT.shape[0]
    steps = 8
    bu = nu // (2 * steps)
    bv = nv // (2 * steps)
    Pt, Qt = pl.pallas_call(
        _fused_kernel,
        grid=(steps,),
        in_specs=[
            pl.BlockSpec((k, d), lambda i: (0, 0)),
            pl.BlockSpec((k, d), lambda i: (0, 0)),
            pl.BlockSpec((d, bu), lambda i: (0, 2 * i)),
            pl.BlockSpec((d, bu), lambda i: (0, 2 * i + 1)),
            pl.BlockSpec((d, bv), lambda i: (0, 2 * i)),
            pl.BlockSpec((d, bv), lambda i: (0, 2 * i + 1)),
        ],
        out_specs=[
            pl.BlockSpec((k, 2 * bu), lambda i: (0, i)),
            pl.BlockSpec((k, 2 * bv), lambda i: (0, i)),
        ],
        out_shape=[
            jax.ShapeDtypeStruct((k, nu), jnp.float32),
            jax.ShapeDtypeStruct((k, nv), jnp.float32),
        ],
    )(EuT, EvT, Ut, Ut, Vt, Vt)
    return (Pt.T, Qt.T)


# speedup vs baseline: 1.3061x; 1.0283x over previous
import jax
import jax.numpy as jnp
from jax.experimental import pallas as pl


def _fused_kernel(eut_ref, evt_ref, ut_ref, vt_ref, pt_ref, qt_ref):
    pt_ref[...] = jnp.dot(eut_ref[...], ut_ref[...],
                          preferred_element_type=jnp.float32)
    qt_ref[...] = jnp.dot(evt_ref[...], vt_ref[...],
                          preferred_element_type=jnp.float32)


def kernel(U, V, Eu, Ev):
    Ut, Vt, EuT, EvT = U.T, V.T, Eu.T, Ev.T
    d, nu = Ut.shape
    _, nv = Vt.shape
    k = EuT.shape[0]
    steps = 16
    bu = nu // steps
    bv = nv // steps
    Pt, Qt = pl.pallas_call(
        _fused_kernel,
        grid=(steps,),
        in_specs=[
            pl.BlockSpec((k, d), lambda i: (0, 0)),
            pl.BlockSpec((k, d), lambda i: (0, 0)),
            pl.BlockSpec((d, bu), lambda i: (0, i)),
            pl.BlockSpec((d, bv), lambda i: (0, i)),
        ],
        out_specs=[
            pl.BlockSpec((k, bu), lambda i: (0, i)),
            pl.BlockSpec((k, bv), lambda i: (0, i)),
        ],
        out_shape=[
            jax.ShapeDtypeStruct((k, nu), jnp.float32),
            jax.ShapeDtypeStruct((k, nv), jnp.float32),
        ],
    )(EuT, EvT, Ut, Vt)
    return (Pt.T, Qt.T)


# speedup vs baseline: 1.3086x; 1.0019x over previous
import jax
import jax.numpy as jnp
from jax.experimental import pallas as pl
from jax.experimental.pallas import tpu as pltpu


def _fused_kernel(eut_ref, evt_ref, ut_ref, vt_ref, pt_ref, qt_ref):
    pt_ref[...] = jnp.dot(eut_ref[...], ut_ref[...],
                          preferred_element_type=jnp.float32)
    qt_ref[...] = jnp.dot(evt_ref[...], vt_ref[...],
                          preferred_element_type=jnp.float32)


def kernel(U, V, Eu, Ev):
    Ut, Vt, EuT, EvT = U.T, V.T, Eu.T, Ev.T
    d, nu = Ut.shape
    _, nv = Vt.shape
    k = EuT.shape[0]
    steps = 16
    bu = nu // steps
    bv = nv // steps
    Pt, Qt = pl.pallas_call(
        _fused_kernel,
        grid=(steps,),
        in_specs=[
            pl.BlockSpec((k, d), lambda i: (0, 0)),
            pl.BlockSpec((k, d), lambda i: (0, 0)),
            pl.BlockSpec((d, bu), lambda i: (0, i)),
            pl.BlockSpec((d, bv), lambda i: (0, i)),
        ],
        out_specs=[
            pl.BlockSpec((k, bu), lambda i: (0, i)),
            pl.BlockSpec((k, bv), lambda i: (0, i)),
        ],
        out_shape=[
            jax.ShapeDtypeStruct((k, nu), jnp.float32),
            jax.ShapeDtypeStruct((k, nv), jnp.float32),
        ],
        compiler_params=pltpu.CompilerParams(
            dimension_semantics=(pltpu.ARBITRARY,),
            disable_bounds_checks=True,
        ),
    )(EuT, EvT, Ut, Vt)
    return (Pt.T, Qt.T)
